# Initial kernel scaffold; baseline (speedup 1.0000x reference)
import jax
import jax.numpy as jnp
from jax.experimental import pallas as pl


def kernel(x, edge_index, edge_type, edge_weight, W_tau, query):
    raise NotImplementedError("write your pallas kernel here")


# speedup vs baseline: 5.6368x; 5.6368x over previous
import functools

import numpy as np
import jax
import jax.numpy as jnp
from jax import lax
from jax.experimental import pallas as pl
from jax.experimental.pallas import tpu as pltpu
from jax.experimental.pallas import tpu_sc as plsc

N = 10000
NP = 10240
E = 160000
M = E + N
R = 8
H = 4
D = 128
EPS = 1e-10
NEG_SLOPE = 0.2

NW = 32
EPW = 5376
MP = NW * EPW
EB = 64
NB = EPW // EB
ROWS_PER_TILE = NP // 16


def _tc_hidden_body(x_ref, w_ref, qi_ref, qo_ref, hid_ref, ain_ref, aout_ref):
    h = lax.dot_general(x_ref[...], w_ref[0],
                        (((1,), (1,)), ((), ())),
                        preferred_element_type=jnp.float32)
    hid_ref[...] = h[None]
    ain_ref[...] = lax.dot_general(h, qi_ref[0],
                                   (((1,), (0,)), ((), ())),
                                   preferred_element_type=jnp.float32)[None]
    aout_ref[...] = lax.dot_general(h, qo_ref[0],
                                    (((1,), (0,)), ((), ())),
                                    preferred_element_type=jnp.float32)[None]


def _tc_epilogue_body(vp_ref, sp_ref, mden_ref, out_ref):
    v = vp_ref[0] + vp_ref[1]
    s = sp_ref[0] + sp_ref[1]
    den = lax.dot_general(s, mden_ref[...],
                          (((1,), (0,)), ((), ())),
                          preferred_element_type=jnp.float32)
    out_ref[...] = jnp.maximum(v / den, 0.0)


def _sc_edge_kernel(idxs_hbm, idxd_hbm, dst_hbm, ewx_hbm, bin_hbm, bout_hbm,
                    h_hbm,
                    vout_hbm, sout_hbm,
                    dst2d, idxs2d, idxd2d,
                    binv, boutv, stage, rows, ewb,
                    vacc, sacc, semb, semr):
    cid = lax.axis_index("c")
    sid = lax.axis_index("s")
    wid = sid * 2 + cid

    zf16 = jnp.zeros((16,), jnp.float32)

    def _zero_rows(j, _):
        for c in range(8):
            rows[j, pl.ds(c * 16, 16)] = zf16
        stage[j, pl.ds(0, 16)] = zf16
        return _
    lax.fori_loop(0, EB, _zero_rows, None)
    for k in range(ROWS_PER_TILE // EB):
        r0 = sid * ROWS_PER_TILE + k * EB
        pltpu.sync_copy(rows, vacc.at[pl.ds(r0, EB)])
        pltpu.sync_copy(stage, sacc.at[pl.ds(r0, EB)])
    plsc.subcore_barrier()

    pltpu.sync_copy(dst_hbm.at[wid], dst2d)
    pltpu.sync_copy(idxs_hbm.at[wid], idxs2d)
    pltpu.sync_copy(idxd_hbm.at[wid], idxd2d)

    def _batch(j, _):
        cpb = pltpu.async_copy(bin_hbm.at[idxs2d.at[j]], binv, semb)
        cpo = pltpu.async_copy(bout_hbm.at[idxd2d.at[j]], boutv, semb)
        cpr = pltpu.async_copy(h_hbm.at[idxs2d.at[j]], rows, semr)
        pltpu.sync_copy(ewx_hbm.at[wid, j], ewb)
        cpb.wait()
        cpo.wait()

        cpr.wait()

        def _edge(e, __):
            u = binv[e, pl.ds(0, 16)] + boutv[e, pl.ds(0, 16)]
            u = jnp.where(u >= 0.0, u, u * NEG_SLOPE)
            val = jnp.exp(u) * ewb[e, pl.ds(0, 16)]
            stage[e, pl.ds(0, 16)] = val
            dn = lax.GatherDimensionNumbers(offset_dims=(),
                                            collapsed_slice_dims=(0,),
                                            start_index_map=(0,))
            for v in range(8):
                s16 = lax.gather(val, jnp.full((16, 1), v // 2, jnp.int32),
                                 dn, (1,),
                                 mode=lax.GatherScatterMode.PROMISE_IN_BOUNDS)
                sl = pl.ds(v * 16, 16)
                rows[e, sl] = rows[e, sl] * s16
            return __
        lax.fori_loop(0, EB, _edge, None)
        pltpu.sync_copy(stage, sacc.at[dst2d.at[j]], add=True)
        pltpu.sync_copy(rows, vacc.at[dst2d.at[j]], add=True)
        return _
    lax.fori_loop(0, NB, _batch, None)

    plsc.subcore_barrier()

    for k in range(ROWS_PER_TILE // EB):
        r0 = sid * ROWS_PER_TILE + k * EB
        pltpu.sync_copy(vacc.at[pl.ds(r0, EB)], rows)
        pltpu.sync_copy(rows, vout_hbm.at[cid, pl.ds(r0, EB)])
        pltpu.sync_copy(sacc.at[pl.ds(r0, EB)], stage)
        pltpu.sync_copy(stage, sout_hbm.at[cid, pl.ds(r0, EB)])


_sc_edge = functools.partial(
    pl.kernel,
    mesh=plsc.VectorSubcoreMesh(core_axis_name="c", subcore_axis_name="s"),
    compiler_params=pltpu.CompilerParams(use_tc_tiling_on_sc=False),
    out_type=[jax.ShapeDtypeStruct((2, NP, D), jnp.float32),
              jax.ShapeDtypeStruct((2, NP, 16), jnp.float32)],
    scratch_types=[
        pltpu.VMEM((NB, EB), jnp.int32),
        pltpu.VMEM((NB, EB), jnp.int32),
        pltpu.VMEM((NB, EB), jnp.int32),
        pltpu.VMEM((EB, 16), jnp.float32),
        pltpu.VMEM((EB, 16), jnp.float32),
        pltpu.VMEM((EB, 16), jnp.float32),
        pltpu.VMEM((EB, 128), jnp.float32),
        pltpu.VMEM((EB, 16), jnp.float32),
        pltpu.VMEM_SHARED((NP, D), jnp.float32),
        pltpu.VMEM_SHARED((NP, 16), jnp.float32),
        pltpu.SemaphoreType.DMA,
        pltpu.SemaphoreType.DMA,
    ],
)(_sc_edge_kernel)


@jax.jit
def kernel(x, edge_index, edge_type, edge_weight, W_tau, query):
    xp = jnp.pad(x, ((0, NP - N), (0, 0)))

    qin = query[:, :, 0::2].reshape(R + 1, D)
    qout = query[:, :, 1::2].reshape(R + 1, D)
    headmask = (jnp.arange(D)[:, None] // (D // H)
                == jnp.arange(16)[None, :]).astype(jnp.float32)
    q2i = qin[:, :, None] * headmask[None]
    q2o = qout[:, :, None] * headmask[None]

    self_idx = jnp.arange(N, dtype=jnp.int32)
    pad = MP - M
    src = jnp.concatenate([edge_index[0], self_idx,
                           jnp.zeros((pad,), jnp.int32)])
    dstf = jnp.concatenate([edge_index[1], self_idx,
                            jnp.full((pad,), N, jnp.int32)])
    relf = jnp.concatenate([edge_type, jnp.full((N,), R, jnp.int32),
                            jnp.zeros((pad,), jnp.int32)])
    idxs = (relf * NP + src).reshape(NW, NB, EB)
    idxd = (relf * NP + dstf).reshape(NW, NB, EB)
    dst = dstf.reshape(NW, NB, EB)
    ew = jnp.concatenate([edge_weight, jnp.ones((N,), jnp.float32),
                          jnp.zeros((pad,), jnp.float32)])
    lanes4 = (jnp.arange(16) < H).astype(jnp.float32)
    lanec = (jnp.arange(16) == H).astype(jnp.float32)
    ewx = (ew[:, None] * lanes4[None, :]
           + lanec[None, :]).reshape(NW, NB, EB, 16)

    nblk = 256
    hidden, ain, aout = pl.pallas_call(
        _tc_hidden_body,
        grid=(R + 1, NP // nblk),
        in_specs=[
            pl.BlockSpec((nblk, D), lambda r, n: (n, 0)),
            pl.BlockSpec((1, D, D), lambda r, n: (r, 0, 0)),
            pl.BlockSpec((1, D, 16), lambda r, n: (r, 0, 0)),
            pl.BlockSpec((1, D, 16), lambda r, n: (r, 0, 0)),
        ],
        out_specs=[
            pl.BlockSpec((1, nblk, D), lambda r, n: (r, n, 0)),
            pl.BlockSpec((1, nblk, 16), lambda r, n: (r, n, 0)),
            pl.BlockSpec((1, nblk, 16), lambda r, n: (r, n, 0)),
        ],
        out_shape=[
            jax.ShapeDtypeStruct((R + 1, NP, D), jnp.float32),
            jax.ShapeDtypeStruct((R + 1, NP, 16), jnp.float32),
            jax.ShapeDtypeStruct((R + 1, NP, 16), jnp.float32),
        ],
    )(xp, W_tau, q2i, q2o)

    h_flat = hidden.reshape((R + 1) * NP, D)
    bin_flat = ain.reshape((R + 1) * NP, 16)
    bout_flat = aout.reshape((R + 1) * NP, 16)

    vpart, spart = _sc_edge(idxs, idxd, dst, ewx, bin_flat, bout_flat,
                            h_flat)

    dchunk = jnp.arange(D) // (D // H)
    mden = jnp.zeros((16, D), jnp.float32)
    mden = mden.at[:H].set((dchunk[None, :] == jnp.arange(H)[:, None])
                           .astype(jnp.float32))
    mden = mden.at[H].set(EPS)

    out = pl.pallas_call(
        _tc_epilogue_body,
        grid=(NP // nblk,),
        in_specs=[
            pl.BlockSpec((2, nblk, D), lambda n: (0, n, 0)),
            pl.BlockSpec((2, nblk, 16), lambda n: (0, n, 0)),
            pl.BlockSpec((16, D), lambda n: (0, 0)),
        ],
        out_specs=pl.BlockSpec((nblk, D), lambda n: (n, 0)),
        out_shape=jax.ShapeDtypeStruct((NP, D), jnp.float32),
    )(vpart, spart, mden)

    return out[:N]


# speedup vs baseline: 6.2979x; 1.1173x over previous
import functools

import numpy as np
import jax
import jax.numpy as jnp
from jax import lax
from jax.experimental import pallas as pl
from jax.experimental.pallas import tpu as pltpu
from jax.experimental.pallas import tpu_sc as plsc

N = 10000
NP = 10240
E = 160000
M = E + N
R = 8
H = 4
D = 128
EPS = 1e-10
NEG_SLOPE = 0.2

NW = 32
EPW = 5376
MP = NW * EPW
EB = 64
NB = EPW // EB
ROWS_PER_TILE = NP // 16


def _tc_hidden_body(x_ref, w_ref, qi_ref, qo_ref, hid_ref, ain_ref, aout_ref):
    h = lax.dot_general(x_ref[...], w_ref[0],
                        (((1,), (1,)), ((), ())),
                        preferred_element_type=jnp.float32)
    hid_ref[...] = h[None]
    ain_ref[...] = lax.dot_general(h, qi_ref[0],
                                   (((1,), (0,)), ((), ())),
                                   preferred_element_type=jnp.float32)[None]
    aout_ref[...] = lax.dot_general(h, qo_ref[0],
                                    (((1,), (0,)), ((), ())),
                                    preferred_element_type=jnp.float32)[None]


def _tc_epilogue_body(vp_ref, sp_ref, mden_ref, out_ref):
    v = vp_ref[0] + vp_ref[1]
    s = sp_ref[0] + sp_ref[1]
    den = lax.dot_general(s, mden_ref[...],
                          (((1,), (0,)), ((), ())),
                          preferred_element_type=jnp.float32)
    out_ref[...] = jnp.maximum(v / den, 0.0)


def _sc_edge_kernel(idxs_hbm, idxd_hbm, dst_hbm, ewx_hbm, bin_hbm, bout_hbm,
                    h_hbm,
                    vout_hbm, sout_hbm,
                    dst2d, idxs2d, idxd2d,
                    binv0, binv1, boutv0, boutv1, stage, rows0, rows1, ewb,
                    vacc, sacc, semb0, semb1, semr0, semr1):
    cid = lax.axis_index("c")
    sid = lax.axis_index("s")
    wid = sid * 2 + cid

    zf16 = jnp.zeros((16,), jnp.float32)

    def _zero_rows(j, _):
        for c in range(8):
            rows0[j, pl.ds(c * 16, 16)] = zf16
        stage[j, pl.ds(0, 16)] = zf16
        return _
    lax.fori_loop(0, EB, _zero_rows, None)
    for k in range(ROWS_PER_TILE // EB):
        r0 = sid * ROWS_PER_TILE + k * EB
        pltpu.sync_copy(rows0, vacc.at[pl.ds(r0, EB)])
        pltpu.sync_copy(stage, sacc.at[pl.ds(r0, EB)])
    plsc.subcore_barrier()

    pltpu.sync_copy(dst_hbm.at[wid], dst2d)
    pltpu.sync_copy(idxs_hbm.at[wid], idxs2d)
    pltpu.sync_copy(idxd_hbm.at[wid], idxd2d)

    bufs = ((binv0, boutv0, rows0, semb0, semr0),
            (binv1, boutv1, rows1, semb1, semr1))

    def _issue(j, bi):
        bv, ov, rw, sb, sr = bufs[bi]
        pltpu.async_copy(bin_hbm.at[idxs2d.at[j]], bv, sb)
        pltpu.async_copy(bout_hbm.at[idxd2d.at[j]], ov, sb)
        pltpu.async_copy(h_hbm.at[idxs2d.at[j]], rw, sr)

    def _wait(j, bi):
        bv, ov, rw, sb, sr = bufs[bi]
        pltpu.make_async_copy(bin_hbm.at[idxs2d.at[j]], bv, sb).wait()
        pltpu.make_async_copy(bout_hbm.at[idxd2d.at[j]], ov, sb).wait()
        pltpu.make_async_copy(h_hbm.at[idxs2d.at[j]], rw, sr).wait()

    _issue(0, 0)

    def _pair(jj, _):
        for b in range(2):
            jcur = jj * 2 + b
            jnext = jnp.minimum(jcur + 1, NB - 1)
            _issue(jnext, 1 - b)
            pltpu.sync_copy(ewx_hbm.at[wid, jcur], ewb)
            _wait(jcur, b)
            bv, ov, rw, _sb, _sr = bufs[b]

            def _edge(e, __):
                u = bv[e, pl.ds(0, 16)] + ov[e, pl.ds(0, 16)]
                u = jnp.where(u >= 0.0, u, u * NEG_SLOPE)
                val = jnp.exp(u) * ewb[e, pl.ds(0, 16)]
                stage[e, pl.ds(0, 16)] = val
                dn = lax.GatherDimensionNumbers(offset_dims=(),
                                                collapsed_slice_dims=(0,),
                                                start_index_map=(0,))
                for h in range(4):
                    s16 = lax.gather(val, jnp.full((16, 1), h, jnp.int32),
                                     dn, (1,),
                                     mode=lax.GatherScatterMode.PROMISE_IN_BOUNDS)
                    for v in (2 * h, 2 * h + 1):
                        sl = pl.ds(v * 16, 16)
                        rw[e, sl] = rw[e, sl] * s16
                return __
            lax.fori_loop(0, EB, _edge, None)
            pltpu.sync_copy(stage, sacc.at[dst2d.at[jcur]], add=True)
            pltpu.sync_copy(rw, vacc.at[dst2d.at[jcur]], add=True)
        return _
    lax.fori_loop(0, NB // 2, _pair, None)
    _wait(NB - 1, 0)

    plsc.subcore_barrier()

    for k in range(ROWS_PER_TILE // EB):
        r0 = sid * ROWS_PER_TILE + k * EB
        pltpu.sync_copy(vacc.at[pl.ds(r0, EB)], rows0)
        pltpu.sync_copy(rows0, vout_hbm.at[cid, pl.ds(r0, EB)])
        pltpu.sync_copy(sacc.at[pl.ds(r0, EB)], stage)
        pltpu.sync_copy(stage, sout_hbm.at[cid, pl.ds(r0, EB)])


_sc_edge = functools.partial(
    pl.kernel,
    mesh=plsc.VectorSubcoreMesh(core_axis_name="c", subcore_axis_name="s"),
    compiler_params=pltpu.CompilerParams(use_tc_tiling_on_sc=False),
    out_type=[jax.ShapeDtypeStruct((2, NP, D), jnp.float32),
              jax.ShapeDtypeStruct((2, NP, 16), jnp.float32)],
    scratch_types=[
        pltpu.VMEM((NB, EB), jnp.int32),
        pltpu.VMEM((NB, EB), jnp.int32),
        pltpu.VMEM((NB, EB), jnp.int32),
        pltpu.VMEM((EB, 16), jnp.float32),
        pltpu.VMEM((EB, 16), jnp.float32),
        pltpu.VMEM((EB, 16), jnp.float32),
        pltpu.VMEM((EB, 16), jnp.float32),
        pltpu.VMEM((EB, 16), jnp.float32),
        pltpu.VMEM((EB, 128), jnp.float32),
        pltpu.VMEM((EB, 128), jnp.float32),
        pltpu.VMEM((EB, 16), jnp.float32),
        pltpu.VMEM_SHARED((NP, D), jnp.float32),
        pltpu.VMEM_SHARED((NP, 16), jnp.float32),
        pltpu.SemaphoreType.DMA,
        pltpu.SemaphoreType.DMA,
        pltpu.SemaphoreType.DMA,
        pltpu.SemaphoreType.DMA,
    ],
)(_sc_edge_kernel)


@jax.jit
def kernel(x, edge_index, edge_type, edge_weight, W_tau, query):
    xp = jnp.pad(x, ((0, NP - N), (0, 0)))

    qin = query[:, :, 0::2].reshape(R + 1, D)
    qout = query[:, :, 1::2].reshape(R + 1, D)
    headmask = (jnp.arange(D)[:, None] // (D // H)
                == jnp.arange(16)[None, :]).astype(jnp.float32)
    q2i = qin[:, :, None] * headmask[None]
    q2o = qout[:, :, None] * headmask[None]

    self_idx = jnp.arange(N, dtype=jnp.int32)
    pad = MP - M
    src = jnp.concatenate([edge_index[0], self_idx,
                           jnp.zeros((pad,), jnp.int32)])
    dstf = jnp.concatenate([edge_index[1], self_idx,
                            jnp.full((pad,), N, jnp.int32)])
    relf = jnp.concatenate([edge_type, jnp.full((N,), R, jnp.int32),
                            jnp.zeros((pad,), jnp.int32)])
    idxs = (relf * NP + src).reshape(NW, NB, EB)
    idxd = (relf * NP + dstf).reshape(NW, NB, EB)
    dst = dstf.reshape(NW, NB, EB)
    ew = jnp.concatenate([edge_weight, jnp.ones((N,), jnp.float32),
                          jnp.zeros((pad,), jnp.float32)])
    lanes4 = (jnp.arange(16) < H).astype(jnp.float32)
    lanec = (jnp.arange(16) == H).astype(jnp.float32)
    ewx = (ew[:, None] * lanes4[None, :]
           + lanec[None, :]).reshape(NW, NB, EB, 16)

    nblk = 256
    hidden, ain, aout = pl.pallas_call(
        _tc_hidden_body,
        grid=(R + 1, NP // nblk),
        in_specs=[
            pl.BlockSpec((nblk, D), lambda r, n: (n, 0)),
            pl.BlockSpec((1, D, D), lambda r, n: (r, 0, 0)),
            pl.BlockSpec((1, D, 16), lambda r, n: (r, 0, 0)),
            pl.BlockSpec((1, D, 16), lambda r, n: (r, 0, 0)),
        ],
        out_specs=[
            pl.BlockSpec((1, nblk, D), lambda r, n: (r, n, 0)),
            pl.BlockSpec((1, nblk, 16), lambda r, n: (r, n, 0)),
            pl.BlockSpec((1, nblk, 16), lambda r, n: (r, n, 0)),
        ],
        out_shape=[
            jax.ShapeDtypeStruct((R + 1, NP, D), jnp.float32),
            jax.ShapeDtypeStruct((R + 1, NP, 16), jnp.float32),
            jax.ShapeDtypeStruct((R + 1, NP, 16), jnp.float32),
        ],
    )(xp, W_tau, q2i, q2o)

    h_flat = hidden.reshape((R + 1) * NP, D)
    bin_flat = ain.reshape((R + 1) * NP, 16)
    bout_flat = aout.reshape((R + 1) * NP, 16)

    vpart, spart = _sc_edge(idxs, idxd, dst, ewx, bin_flat, bout_flat,
                            h_flat)

    dchunk = jnp.arange(D) // (D // H)
    mden = jnp.zeros((16, D), jnp.float32)
    mden = mden.at[:H].set((dchunk[None, :] == jnp.arange(H)[:, None])
                           .astype(jnp.float32))
    mden = mden.at[H].set(EPS)

    out = pl.pallas_call(
        _tc_epilogue_body,
        grid=(NP // nblk,),
        in_specs=[
            pl.BlockSpec((2, nblk, D), lambda n: (0, n, 0)),
            pl.BlockSpec((2, nblk, 16), lambda n: (0, n, 0)),
            pl.BlockSpec((16, D), lambda n: (0, 0)),
        ],
        out_specs=pl.BlockSpec((nblk, D), lambda n: (n, 0)),
        out_shape=jax.ShapeDtypeStruct((NP, D), jnp.float32),
    )(vpart, spart, mden)

    return out[:N]
